# baseline (speedup 1.0000x reference)
import jax
import jax.numpy as jnp
from jax.experimental import pallas as pl

SCALE_BOUND = 0.11
LIKELIHOOD_BOUND = 1e-09

_P = 0.3275911
_A1 = 0.254829592
_A2 = -0.284496736
_A3 = 1.421413741
_A4 = -1.453152027
_A5 = 1.061405429
_INV_SQRT2 = 0.7071067811865476


def _erfc_nonneg(a):
    t = 1.0 / (1.0 + _P * a)
    poly = t * (_A1 + t * (_A2 + t * (_A3 + t * (_A4 + t * _A5))))
    return poly * jnp.exp(-(a * a))


def _body(x_ref, s_ref, m_ref, out_ref, lik_ref):
    x = x_ref[...]
    s = s_ref[...]
    m = m_ref[...]
    q = jnp.round(x - m)
    out_ref[...] = q + m
    v = jnp.abs(q)
    sb = jnp.maximum(s, SCALE_BOUND)
    inv = _INV_SQRT2 / sb
    a = (v + 0.5) * inv
    b = (v - 0.5) * inv
    ea = _erfc_nonneg(a)
    eb_mag = _erfc_nonneg(jnp.abs(b))
    eb = jnp.where(b < 0.0, 2.0 - eb_mag, eb_mag)
    lik = 0.5 * (eb - ea)
    lik_ref[...] = jnp.maximum(lik, LIKELIHOOD_BOUND)


def kernel(x, scales, means):
    shape = x.shape
    b, c, h, w = shape
    bc = 24
    grid = (b, c // bc)
    spec = pl.BlockSpec((1, bc, h, w), lambda i, j: (i, j, 0, 0))
    out, lik = pl.pallas_call(
        _body,
        grid=grid,
        in_specs=[spec, spec, spec],
        out_specs=[spec, spec],
        out_shape=[
            jax.ShapeDtypeStruct(shape, jnp.float32),
            jax.ShapeDtypeStruct(shape, jnp.float32),
        ],
    )(x, scales, means)
    return out, lik


# speedup vs baseline: 1.1776x; 1.1776x over previous
import jax
import jax.numpy as jnp
from jax.experimental import pallas as pl

SCALE_BOUND = 0.11
LIKELIHOOD_BOUND = 1e-09

_P = 0.3275911
_A1 = 0.254829592
_A2 = -0.284496736
_A3 = 1.421413741
_A4 = -1.453152027
_A5 = 1.061405429
_INV_SQRT2 = 0.7071067811865476


def _erfc_nonneg(a):
    t = 1.0 / (1.0 + _P * a)
    poly = t * (_A1 + t * (_A2 + t * (_A3 + t * (_A4 + t * _A5))))
    return poly * jnp.exp(-(a * a))


def _body(x_ref, s_ref, m_ref, out_ref, lik_ref):
    x = x_ref[...]
    s = s_ref[...]
    m = m_ref[...]
    q = jnp.round(x - m)
    out_ref[...] = q + m
    v = jnp.abs(q)
    sb = jnp.maximum(s, SCALE_BOUND)
    inv = _INV_SQRT2 / sb
    a = (v + 0.5) * inv
    b = (v - 0.5) * inv
    ea = _erfc_nonneg(a)
    eb_mag = _erfc_nonneg(jnp.abs(b))
    eb = jnp.where(b < 0.0, 2.0 - eb_mag, eb_mag)
    lik = 0.5 * (eb - ea)
    lik_ref[...] = jnp.maximum(lik, LIKELIHOOD_BOUND)


def kernel(x, scales, means):
    shape = x.shape
    b, c, h, w = shape
    bb, bc = 2, 96
    grid = (b // bb, c // bc)
    spec = pl.BlockSpec((bb, bc, h, w), lambda i, j: (i, j, 0, 0))
    out, lik = pl.pallas_call(
        _body,
        grid=grid,
        in_specs=[spec, spec, spec],
        out_specs=[spec, spec],
        out_shape=[
            jax.ShapeDtypeStruct(shape, jnp.float32),
            jax.ShapeDtypeStruct(shape, jnp.float32),
        ],
    )(x, scales, means)
    return out, lik


# speedup vs baseline: 1.9362x; 1.6442x over previous
import jax
import jax.numpy as jnp
from jax.experimental import pallas as pl

SCALE_BOUND = 0.11
LIKELIHOOD_BOUND = 1e-09

_P = 0.3275911
_A1 = 0.254829592
_A2 = -0.284496736
_A3 = 1.421413741
_A4 = -1.453152027
_A5 = 1.061405429
_INV_SQRT2 = 0.7071067811865476


def _erfc_nonneg(a):
    t = 1.0 / (1.0 + _P * a)
    poly = t * (_A1 + t * (_A2 + t * (_A3 + t * (_A4 + t * _A5))))
    return poly * jnp.exp(-(a * a))


def _body(x_ref, s_ref, m_ref, out_ref, lik_ref):
    x = x_ref[...]
    s = s_ref[...]
    m = m_ref[...]
    q = jnp.round(x - m)
    out_ref[...] = q + m
    v = jnp.abs(q)
    sb = jnp.maximum(s, SCALE_BOUND)
    inv = _INV_SQRT2 / sb
    a = (v + 0.5) * inv
    b = (v - 0.5) * inv
    ea = _erfc_nonneg(a)
    eb_mag = _erfc_nonneg(jnp.abs(b))
    eb = jnp.where(b < 0.0, 2.0 - eb_mag, eb_mag)
    lik = 0.5 * (eb - ea)
    lik_ref[...] = jnp.maximum(lik, LIKELIHOOD_BOUND)


def kernel(x, scales, means):
    shape = x.shape
    b, c, h, w = shape
    r4 = (b, c, (h * w) // 128, 128)
    x4 = x.reshape(r4)
    s4 = scales.reshape(r4)
    m4 = means.reshape(r4)
    bc = 24
    grid = (b, c // bc)
    spec = pl.BlockSpec((1, bc, r4[2], 128), lambda i, j: (i, j, 0, 0))
    out, lik = pl.pallas_call(
        _body,
        grid=grid,
        in_specs=[spec, spec, spec],
        out_specs=[spec, spec],
        out_shape=[
            jax.ShapeDtypeStruct(r4, jnp.float32),
            jax.ShapeDtypeStruct(r4, jnp.float32),
        ],
    )(x4, s4, m4)
    return out.reshape(shape), lik.reshape(shape)


# speedup vs baseline: 2.7717x; 1.4315x over previous
import jax
import jax.numpy as jnp
from jax.experimental import pallas as pl

SCALE_BOUND = 0.11
LIKELIHOOD_BOUND = 1e-09

_P = 0.3275911
_A1 = 0.254829592
_A2 = -0.284496736
_A3 = 1.421413741
_A4 = -1.453152027
_A5 = 1.061405429
_INV_SQRT2 = 0.7071067811865476


def _erfc_nonneg(a):
    t = 1.0 / (1.0 + _P * a)
    poly = t * (_A1 + t * (_A2 + t * (_A3 + t * (_A4 + t * _A5))))
    return poly * jnp.exp(-(a * a))


def _body(x_ref, s_ref, m_ref, out_ref, lik_ref):
    x = x_ref[...]
    s = s_ref[...]
    m = m_ref[...]
    q = jnp.round(x - m)
    out_ref[...] = q + m
    v = jnp.abs(q)
    sb = jnp.maximum(s, SCALE_BOUND)
    inv = _INV_SQRT2 / sb
    a = (v + 0.5) * inv
    b = (v - 0.5) * inv
    ea = _erfc_nonneg(a)
    eb_mag = _erfc_nonneg(jnp.abs(b))
    eb = jnp.where(b < 0.0, 2.0 - eb_mag, eb_mag)
    lik = 0.5 * (eb - ea)
    lik_ref[...] = jnp.maximum(lik, LIKELIHOOD_BOUND)


def kernel(x, scales, means):
    shape = x.shape
    b, c, h, w = shape
    r4 = (b, c, (h * w) // 128, 128)
    x4 = x.reshape(r4)
    s4 = scales.reshape(r4)
    m4 = means.reshape(r4)
    bc = 192
    grid = (b, c // bc)
    spec = pl.BlockSpec((1, bc, r4[2], 128), lambda i, j: (i, j, 0, 0))
    out, lik = pl.pallas_call(
        _body,
        grid=grid,
        in_specs=[spec, spec, spec],
        out_specs=[spec, spec],
        out_shape=[
            jax.ShapeDtypeStruct(r4, jnp.float32),
            jax.ShapeDtypeStruct(r4, jnp.float32),
        ],
    )(x4, s4, m4)
    return out.reshape(shape), lik.reshape(shape)
